# Initial kernel scaffold; baseline (speedup 1.0000x reference)
import jax
import jax.numpy as jnp
from jax.experimental import pallas as pl


def kernel(x, edge_index, W1l, W1r, b1, W2l, W2r, b2):
    raise NotImplementedError("write your pallas kernel here")


# speedup vs baseline: 13.8221x; 13.8221x over previous
import functools

import jax
import jax.numpy as jnp
from jax import lax
from jax.experimental import pallas as pl
from jax.experimental.pallas import tpu as pltpu
from jax.experimental.pallas import tpu_sc as plsc

_N = 10000
_E = 320000
_D_IN = 128
_HID = 8
_D_OUT = 128

_NC = 2
_NS = 16
_NW = _NC * _NS
_BATCH = 128
_INNER = 16
_CHUNK = _BATCH * _INNER
_STEPS = 5
_EPW = _CHUNK * _STEPS
_E_PAD = _EPW * _NW
_VROW = 16
_NROW = 10112
_SLAB = _NROW // _NS


def _segsum_body(vals_hbm, srcr_hbm, dstr_hbm, out_hbm,
                 src_v, dst_v, rows_v, acc_sh, sem):
    c = lax.axis_index("c")
    s = lax.axis_index("s")
    wid = c * _NS + s

    def _zero_row(i, carry):
        rows_v[i, :] = jnp.zeros((_VROW,), jnp.float32)
        return carry
    lax.fori_loop(0, _SLAB, _zero_row, 0)
    pltpu.sync_copy(rows_v.at[pl.ds(0, _SLAB)],
                    acc_sh.at[pl.ds(s * _SLAB, _SLAB)])
    plsc.subcore_barrier()

    base_row = wid * (_EPW // _BATCH)

    def _step(g, carry):
        r0 = base_row + g * _INNER
        pltpu.sync_copy(srcr_hbm.at[pl.ds(r0, _INNER)], src_v)
        pltpu.sync_copy(dstr_hbm.at[pl.ds(r0, _INNER)], dst_v)
        for j in range(_INNER):
            rows_j = rows_v.at[pl.ds(j * _BATCH, _BATCH)]
            pltpu.async_copy(vals_hbm.at[src_v.at[j]], rows_j, sem).wait()
            pltpu.sync_copy(rows_j, acc_sh.at[dst_v.at[j]], add=True)
        return carry

    lax.fori_loop(0, _STEPS, _step, 0)
    plsc.subcore_barrier()

    pltpu.sync_copy(acc_sh.at[pl.ds(s * _SLAB, _SLAB)],
                    out_hbm.at[c, pl.ds(s * _SLAB, _SLAB)])


@functools.cache
def _make_segsum():
    return pl.kernel(
        _segsum_body,
        out_type=jax.ShapeDtypeStruct((_NC, _NROW, _VROW), jnp.float32),
        mesh=plsc.VectorSubcoreMesh(core_axis_name="c", subcore_axis_name="s",
                                    num_cores=_NC, num_subcores=_NS),
        scratch_types=[
            pltpu.VMEM((_INNER, _BATCH), jnp.int32),
            pltpu.VMEM((_INNER, _BATCH), jnp.int32),
            pltpu.VMEM((_CHUNK, _VROW), jnp.float32),
            pltpu.VMEM_SHARED((_NROW, _VROW), jnp.float32),
            pltpu.SemaphoreType.DMA,
        ],
        compiler_params=pltpu.CompilerParams(use_tc_tiling_on_sc=False),
    )


def _proj_body(x_ref, w1l_ref, w1r_ref, y1p_ref, xw1r_ref):
    x = x_ref[...]
    y = jnp.dot(x, w1l_ref[...], preferred_element_type=jnp.float32,
                precision=lax.Precision.HIGHEST)
    ones = jnp.ones((_N, 1), jnp.float32)
    zeros = jnp.zeros((_N, _VROW - _HID - 1), jnp.float32)
    y1p_ref[...] = jnp.concatenate([y, ones, zeros], axis=1)
    xw1r_ref[...] = jnp.dot(x, w1r_ref[...], preferred_element_type=jnp.float32,
                            precision=lax.Precision.HIGHEST)


def _proj(x, w1l, w1r):
    return pl.pallas_call(
        _proj_body,
        out_shape=[jax.ShapeDtypeStruct((_N, _VROW), jnp.float32),
                   jax.ShapeDtypeStruct((_N, _HID), jnp.float32)],
    )(x, w1l, w1r)


def _hidden_body(agg_ref, xw1r_ref, b1_ref, h16_ref):
    p = (agg_ref[0] + agg_ref[1])[:_N]
    cnt = jnp.maximum(p[:, _HID], 1.0)
    mean = p[:, :_HID] / cnt[:, None]
    h = jnp.maximum(mean + xw1r_ref[...] + b1_ref[...], 0.0)
    ones = jnp.ones((_N, 1), jnp.float32)
    zeros = jnp.zeros((_N, _VROW - _HID - 1), jnp.float32)
    h16_ref[...] = jnp.concatenate([h, ones, zeros], axis=1)


def _hidden(agg1, xw1r, b1):
    return pl.pallas_call(
        _hidden_body,
        out_shape=jax.ShapeDtypeStruct((_N, _VROW), jnp.float32),
    )(agg1, xw1r, b1)


def _out_body(agg_ref, h16_ref, w2l_ref, w2r_ref, b2_ref, logp_ref, h2_ref):
    p = (agg_ref[0] + agg_ref[1])[:_N]
    cnt = jnp.maximum(p[:, _HID], 1.0)
    mean = p[:, :_HID] / cnt[:, None]
    h = h16_ref[:, :_HID]
    h2 = (jnp.dot(mean, w2l_ref[...], preferred_element_type=jnp.float32,
                  precision=lax.Precision.HIGHEST)
          + jnp.dot(h, w2r_ref[...], preferred_element_type=jnp.float32,
                    precision=lax.Precision.HIGHEST)
          + b2_ref[...])
    h2_ref[...] = h2
    m = jnp.max(h2, axis=1, keepdims=True)
    lse = jnp.log(jnp.sum(jnp.exp(h2 - m), axis=1, keepdims=True)) + m
    logp_ref[...] = h2 - lse


def _out(agg2, h16, w2l, w2r, b2):
    return pl.pallas_call(
        _out_body,
        out_shape=[jax.ShapeDtypeStruct((_N, _D_OUT), jnp.float32),
                   jax.ShapeDtypeStruct((_N, _D_OUT), jnp.float32)],
        compiler_params=pltpu.CompilerParams(
            vmem_limit_bytes=100 * 1024 * 1024),
    )(agg2, h16, w2l, w2r, b2)


def kernel(x, edge_index, W1l, W1r, b1, W2l, W2r, b2):
    src = edge_index[0]
    dst = edge_index[1]
    npad = _E_PAD - _E
    pad_src = (jnp.arange(npad, dtype=jnp.int32) * 61) % _N
    pad_dst = _N + (jnp.arange(npad, dtype=jnp.int32) % (_NROW - _N))
    srcr = jnp.concatenate([src, pad_src]).reshape(_E_PAD // _BATCH, _BATCH)
    dstr = jnp.concatenate([dst, pad_dst]).reshape(_E_PAD // _BATCH, _BATCH)

    segsum = _make_segsum()
    y1p, xw1r = _proj(x, W1l, W1r)
    agg1 = segsum(y1p, srcr, dstr)
    h16 = _hidden(agg1, xw1r, b1.reshape(1, _HID))
    agg2 = segsum(h16, srcr, dstr)
    logp, h2 = _out(agg2, h16, W2l, W2r, b2.reshape(1, _D_OUT))
    return (logp, h2)


# speedup vs baseline: 23.6413x; 1.7104x over previous
import functools

import jax
import jax.numpy as jnp
from jax import lax
from jax.experimental import pallas as pl
from jax.experimental.pallas import tpu as pltpu
from jax.experimental.pallas import tpu_sc as plsc

_N = 10000
_E = 320000
_D_IN = 128
_HID = 8
_D_OUT = 128

_NC = 2
_NS = 16
_NW = _NC * _NS
_BATCH = 128
_G = 4
_NB = 80
_NGRP = _NB // _G
_GB = _G * _BATCH
_EPW = _NB * _BATCH
_E_PAD = _EPW * _NW
_VROW = 16
_NROW = 10112
_SLAB = _NROW // _NS


def _segsum_body(vals_hbm, srcr_hbm, dstr_hbm, zeros_hbm, out_hbm,
                 src_v, dst_v, rows_v, acc_sh, gsem0, gsem1, ssem0, ssem1):
    c = lax.axis_index("c")
    s = lax.axis_index("s")
    wid = c * _NS + s

    pltpu.sync_copy(zeros_hbm, acc_sh.at[pl.ds(s * _SLAB, _SLAB)])

    base_row = wid * _NB
    pltpu.sync_copy(srcr_hbm.at[pl.ds(base_row, _NB)], src_v)
    pltpu.sync_copy(dstr_hbm.at[pl.ds(base_row, _NB)], dst_v)

    def fire_gathers(g, p, sem):
        for j in range(_G):
            pltpu.async_copy(vals_hbm.at[src_v.at[g * _G + j]],
                             rows_v.at[p, pl.ds(j * _BATCH, _BATCH)], sem)

    def fire_scatters(g, p, sem):
        for j in range(_G):
            pltpu.async_copy(rows_v.at[p, pl.ds(j * _BATCH, _BATCH)],
                             acc_sh.at[dst_v.at[g * _G + j]], sem, add=True)

    def drain(p, sem):
        pltpu.make_async_copy(vals_hbm.at[pl.ds(0, _GB)],
                              rows_v.at[p], sem).wait()

    fire_gathers(0, 0, gsem0)
    fire_gathers(1, 1, gsem1)
    plsc.subcore_barrier()

    def _pipe(i, carry):
        g = 2 * i
        drain(0, gsem0)
        fire_scatters(g, 0, ssem0)
        drain(1, gsem1)
        fire_scatters(g + 1, 1, ssem1)
        drain(0, ssem0)

        @pl.when(g + 2 < _NGRP)
        def _():
            fire_gathers(g + 2, 0, gsem0)
        drain(1, ssem1)

        @pl.when(g + 3 < _NGRP)
        def _():
            fire_gathers(g + 3, 1, gsem1)
        return carry

    lax.fori_loop(0, _NGRP // 2, _pipe, 0)
    plsc.subcore_barrier()

    pltpu.sync_copy(acc_sh.at[pl.ds(s * _SLAB, _SLAB)],
                    out_hbm.at[c, pl.ds(s * _SLAB, _SLAB)])


@functools.cache
def _make_segsum():
    return pl.kernel(
        _segsum_body,
        out_type=jax.ShapeDtypeStruct((_NC, _NROW, _VROW), jnp.float32),
        mesh=plsc.VectorSubcoreMesh(core_axis_name="c", subcore_axis_name="s",
                                    num_cores=_NC, num_subcores=_NS),
        scratch_types=[
            pltpu.VMEM((_NB, _BATCH), jnp.int32),
            pltpu.VMEM((_NB, _BATCH), jnp.int32),
            pltpu.VMEM((2, _GB, _VROW), jnp.float32),
            pltpu.VMEM_SHARED((_NROW, _VROW), jnp.float32),
            pltpu.SemaphoreType.DMA,
            pltpu.SemaphoreType.DMA,
            pltpu.SemaphoreType.DMA,
            pltpu.SemaphoreType.DMA,
        ],
        compiler_params=pltpu.CompilerParams(use_tc_tiling_on_sc=False),
    )


def _proj_body(x_ref, w1l_ref, w1r_ref, y1p_ref, xw1r_ref):
    x = x_ref[...]
    y = jnp.dot(x, w1l_ref[...], preferred_element_type=jnp.float32,
                precision=lax.Precision.HIGHEST)
    ones = jnp.ones((_N, 1), jnp.float32)
    zeros = jnp.zeros((_N, _VROW - _HID - 1), jnp.float32)
    y1p_ref[...] = jnp.concatenate([y, ones, zeros], axis=1)
    xw1r_ref[...] = jnp.dot(x, w1r_ref[...], preferred_element_type=jnp.float32,
                            precision=lax.Precision.HIGHEST)


def _proj(x, w1l, w1r):
    return pl.pallas_call(
        _proj_body,
        out_shape=[jax.ShapeDtypeStruct((_N, _VROW), jnp.float32),
                   jax.ShapeDtypeStruct((_N, _HID), jnp.float32)],
    )(x, w1l, w1r)


def _hidden_body(agg_ref, xw1r_ref, b1_ref, h16_ref):
    p = (agg_ref[0] + agg_ref[1])[:_N]
    cnt = jnp.maximum(p[:, _HID], 1.0)
    mean = p[:, :_HID] / cnt[:, None]
    h = jnp.maximum(mean + xw1r_ref[...] + b1_ref[...], 0.0)
    ones = jnp.ones((_N, 1), jnp.float32)
    zeros = jnp.zeros((_N, _VROW - _HID - 1), jnp.float32)
    h16_ref[...] = jnp.concatenate([h, ones, zeros], axis=1)


def _hidden(agg1, xw1r, b1):
    return pl.pallas_call(
        _hidden_body,
        out_shape=jax.ShapeDtypeStruct((_N, _VROW), jnp.float32),
    )(agg1, xw1r, b1)


def _out_body(agg_ref, h16_ref, w2l_ref, w2r_ref, b2_ref, logp_ref, h2_ref):
    p = (agg_ref[0] + agg_ref[1])[:_N]
    cnt = jnp.maximum(p[:, _HID], 1.0)
    mean = p[:, :_HID] / cnt[:, None]
    h = h16_ref[:, :_HID]
    h2 = (jnp.dot(mean, w2l_ref[...], preferred_element_type=jnp.float32,
                  precision=lax.Precision.HIGHEST)
          + jnp.dot(h, w2r_ref[...], preferred_element_type=jnp.float32,
                    precision=lax.Precision.HIGHEST)
          + b2_ref[...])
    h2_ref[...] = h2
    m = jnp.max(h2, axis=1, keepdims=True)
    lse = jnp.log(jnp.sum(jnp.exp(h2 - m), axis=1, keepdims=True)) + m
    logp_ref[...] = h2 - lse


def _out(agg2, h16, w2l, w2r, b2):
    return pl.pallas_call(
        _out_body,
        out_shape=[jax.ShapeDtypeStruct((_N, _D_OUT), jnp.float32),
                   jax.ShapeDtypeStruct((_N, _D_OUT), jnp.float32)],
        compiler_params=pltpu.CompilerParams(
            vmem_limit_bytes=100 * 1024 * 1024),
    )(agg2, h16, w2l, w2r, b2)


def kernel(x, edge_index, W1l, W1r, b1, W2l, W2r, b2):
    src = edge_index[0]
    dst = edge_index[1]
    npad = _E_PAD - _E
    pad_src = (jnp.arange(npad, dtype=jnp.int32) * 61) % _N
    pad_dst = _N + (jnp.arange(npad, dtype=jnp.int32) % (_NROW - _N))
    srcr = jnp.concatenate([src, pad_src]).reshape(_E_PAD // _BATCH, _BATCH)
    dstr = jnp.concatenate([dst, pad_dst]).reshape(_E_PAD // _BATCH, _BATCH)

    segsum = _make_segsum()
    zeros = jnp.zeros((_SLAB, _VROW), jnp.float32)
    y1p, xw1r = _proj(x, W1l, W1r)
    agg1 = segsum(y1p, srcr, dstr, zeros)
    h16 = _hidden(agg1, xw1r, b1.reshape(1, _HID))
    agg2 = segsum(h16, srcr, dstr, zeros)
    logp, h2 = _out(agg2, h16, W2l, W2r, b2.reshape(1, _D_OUT))
    return (logp, h2)


# speedup vs baseline: 25.0481x; 1.0595x over previous
import functools

import jax
import jax.numpy as jnp
from jax import lax
from jax.experimental import pallas as pl
from jax.experimental.pallas import tpu as pltpu
from jax.experimental.pallas import tpu_sc as plsc

_N = 10000
_E = 320000
_D_IN = 128
_HID = 8
_D_OUT = 128

_NC = 2
_NS = 16
_NW = _NC * _NS
_BATCH = 128
_G = 4
_NB = 80
_NGRP = _NB // _G
_GB = _G * _BATCH
_EPW = _NB * _BATCH
_E_PAD = _EPW * _NW
_VROW = 16
_NROW = 10112
_SLAB = _NROW // _NS


def _segsum_body(vals_hbm, srcr_hbm, dstr_hbm, zeros_hbm, out_hbm,
                 src_v, dst_v, rows_v, acc_sh, gsem0, gsem1, ssem0, ssem1):
    c = lax.axis_index("c")
    s = lax.axis_index("s")
    wid = c * _NS + s

    pltpu.sync_copy(zeros_hbm, acc_sh.at[pl.ds(s * _SLAB, _SLAB)])

    base_row = wid * _NB
    pltpu.sync_copy(srcr_hbm.at[pl.ds(base_row, _NB)], src_v)
    pltpu.sync_copy(dstr_hbm.at[pl.ds(base_row, _NB)], dst_v)

    def fire_gathers(g, p, sem):
        for j in range(_G):
            pltpu.async_copy(vals_hbm.at[src_v.at[g * _G + j]],
                             rows_v.at[p, pl.ds(j * _BATCH, _BATCH)], sem)

    def fire_scatters(g, p, sem):
        for j in range(_G):
            pltpu.async_copy(rows_v.at[p, pl.ds(j * _BATCH, _BATCH)],
                             acc_sh.at[dst_v.at[g * _G + j]], sem, add=True)

    def drain(p, sem):
        pltpu.make_async_copy(vals_hbm.at[pl.ds(0, _GB)],
                              rows_v.at[p], sem).wait()

    fire_gathers(0, 0, gsem0)
    fire_gathers(1, 1, gsem1)
    plsc.subcore_barrier()

    def _pipe(i, carry):
        g = 2 * i
        drain(0, gsem0)
        fire_scatters(g, 0, ssem0)
        drain(1, gsem1)
        fire_scatters(g + 1, 1, ssem1)
        drain(0, ssem0)

        @pl.when(g + 2 < _NGRP)
        def _():
            fire_gathers(g + 2, 0, gsem0)
        drain(1, ssem1)

        @pl.when(g + 3 < _NGRP)
        def _():
            fire_gathers(g + 3, 1, gsem1)
        return carry

    lax.fori_loop(0, _NGRP // 2, _pipe, 0)
    plsc.subcore_barrier()

    pltpu.sync_copy(acc_sh.at[pl.ds(s * _SLAB, _SLAB)],
                    out_hbm.at[c, pl.ds(s * _SLAB, _SLAB)])


@functools.cache
def _make_segsum():
    return pl.kernel(
        _segsum_body,
        out_type=jax.ShapeDtypeStruct((_NC, _NROW, _VROW), jnp.float32),
        mesh=plsc.VectorSubcoreMesh(core_axis_name="c", subcore_axis_name="s",
                                    num_cores=_NC, num_subcores=_NS),
        scratch_types=[
            pltpu.VMEM((_NB, _BATCH), jnp.int32),
            pltpu.VMEM((_NB, _BATCH), jnp.int32),
            pltpu.VMEM((2, _GB, _VROW), jnp.float32),
            pltpu.VMEM_SHARED((_NROW, _VROW), jnp.float32),
            pltpu.SemaphoreType.DMA,
            pltpu.SemaphoreType.DMA,
            pltpu.SemaphoreType.DMA,
            pltpu.SemaphoreType.DMA,
        ],
        compiler_params=pltpu.CompilerParams(use_tc_tiling_on_sc=False),
    )


_RB = 1000


def _proj_body(x_ref, w_ref, y1p_ref, xw1r_ref):
    y = jnp.dot(x_ref[...], w_ref[...], preferred_element_type=jnp.float32,
                precision=lax.Precision.DEFAULT)
    ones = jnp.ones((_RB, 1), jnp.float32)
    zeros = jnp.zeros((_RB, _VROW - _HID - 1), jnp.float32)
    y1p_ref[...] = jnp.concatenate([y[:, :_HID], ones, zeros], axis=1)
    xw1r_ref[...] = y[:, _HID:]


def _proj(x, w_cat):
    return pl.pallas_call(
        _proj_body,
        grid=(_N // _RB,),
        in_specs=[pl.BlockSpec((_RB, _D_IN), lambda i: (i, 0)),
                  pl.BlockSpec((_D_IN, 2 * _HID), lambda i: (0, 0))],
        out_specs=[pl.BlockSpec((_RB, _VROW), lambda i: (i, 0)),
                   pl.BlockSpec((_RB, _HID), lambda i: (i, 0))],
        out_shape=[jax.ShapeDtypeStruct((_N, _VROW), jnp.float32),
                   jax.ShapeDtypeStruct((_N, _HID), jnp.float32)],
    )(x, w_cat)


def _hidden_body(agg_ref, xw1r_ref, b1_ref, h16_ref):
    p = agg_ref[0] + agg_ref[1]
    cnt = jnp.maximum(p[:, _HID], 1.0)
    mean = p[:, :_HID] / cnt[:, None]
    h = jnp.maximum(mean + xw1r_ref[...] + b1_ref[...], 0.0)
    ones = jnp.ones((_RB, 1), jnp.float32)
    zeros = jnp.zeros((_RB, _VROW - _HID - 1), jnp.float32)
    h16_ref[...] = jnp.concatenate([h, ones, zeros], axis=1)


def _hidden(agg1, xw1r, b1):
    return pl.pallas_call(
        _hidden_body,
        grid=(_N // _RB,),
        in_specs=[pl.BlockSpec((_NC, _RB, _VROW), lambda i: (0, i, 0)),
                  pl.BlockSpec((_RB, _HID), lambda i: (i, 0)),
                  pl.BlockSpec((1, _HID), lambda i: (0, 0))],
        out_specs=pl.BlockSpec((_RB, _VROW), lambda i: (i, 0)),
        out_shape=jax.ShapeDtypeStruct((_N, _VROW), jnp.float32),
    )(agg1, xw1r, b1)


def _out_body(agg_ref, h16_ref, w_ref, b2_ref, logp_ref, h2_ref):
    p = agg_ref[0] + agg_ref[1]
    cnt = jnp.maximum(p[:, _HID], 1.0)
    mean = p[:, :_HID] / cnt[:, None]
    mh = jnp.concatenate([mean, h16_ref[:, :_HID]], axis=1)
    h2 = jnp.dot(mh, w_ref[...], preferred_element_type=jnp.float32,
                 precision=lax.Precision.DEFAULT) + b2_ref[...]
    h2_ref[...] = h2
    m = jnp.max(h2, axis=1, keepdims=True)
    lse = jnp.log(jnp.sum(jnp.exp(h2 - m), axis=1, keepdims=True)) + m
    logp_ref[...] = h2 - lse


def _out(agg2, h16, w_cat, b2):
    return pl.pallas_call(
        _out_body,
        grid=(_N // _RB,),
        in_specs=[pl.BlockSpec((_NC, _RB, _VROW), lambda i: (0, i, 0)),
                  pl.BlockSpec((_RB, _VROW), lambda i: (i, 0)),
                  pl.BlockSpec((2 * _HID, _D_OUT), lambda i: (0, 0)),
                  pl.BlockSpec((1, _D_OUT), lambda i: (0, 0))],
        out_specs=[pl.BlockSpec((_RB, _D_OUT), lambda i: (i, 0)),
                   pl.BlockSpec((_RB, _D_OUT), lambda i: (i, 0))],
        out_shape=[jax.ShapeDtypeStruct((_N, _D_OUT), jnp.float32),
                   jax.ShapeDtypeStruct((_N, _D_OUT), jnp.float32)],
    )(agg2, h16, w_cat, b2)


def kernel(x, edge_index, W1l, W1r, b1, W2l, W2r, b2):
    src = edge_index[0]
    dst = edge_index[1]
    npad = _E_PAD - _E
    pad_src = (jnp.arange(npad, dtype=jnp.int32) * 61) % _N
    pad_dst = _N + (jnp.arange(npad, dtype=jnp.int32) % (_NROW - _N))
    srcr = jnp.concatenate([src, pad_src]).reshape(_E_PAD // _BATCH, _BATCH)
    dstr = jnp.concatenate([dst, pad_dst]).reshape(_E_PAD // _BATCH, _BATCH)

    segsum = _make_segsum()
    zeros = jnp.zeros((_SLAB, _VROW), jnp.float32)
    y1p, xw1r = _proj(x, jnp.concatenate([W1l, W1r], axis=1))
    agg1 = segsum(y1p, srcr, dstr, zeros)
    h16 = _hidden(agg1, xw1r, b1.reshape(1, _HID))
    agg2 = segsum(h16, srcr, dstr, zeros)
    logp, h2 = _out(agg2, h16, jnp.concatenate([W2l, W2r], axis=0),
                    b2.reshape(1, _D_OUT))
    return (logp, h2)
